# initial kernel scaffold (unmeasured)
import jax
import jax.numpy as jnp
from jax import lax
from jax.experimental import pallas as pl
from jax.experimental.pallas import tpu as pltpu

N_Z = 4
BLK = 512


def kernel(x, dy, gamma):
    m, d = x.shape
    nsteps = m // BLK

    def body(x_ref, dy_ref, gamma_ref, out_ref, comm_ref, send_sems, recv_sems):
        i = pl.program_id(0)

        @pl.when(i == 0)
        def _():
            out_ref[...] = jnp.zeros_like(out_ref)

        xv = x_ref[...]
        mu = jnp.mean(xv, axis=1, keepdims=True)
        xc = xv - mu
        var = jnp.mean(xc * xc, axis=1, keepdims=True)
        xhat = xc * lax.rsqrt(var + 1e-5)
        dyv = dy_ref[...]
        out_ref[0, :] += jnp.sum(dyv * xhat, axis=0)
        out_ref[1, :] += jnp.sum(dyv, axis=0)

        @pl.when(i == nsteps - 1)
        def _():
            my_x = lax.axis_index("x")
            my_y = lax.axis_index("y")
            my_z = lax.axis_index("z")
            left = (my_x, my_y, (my_z - 1) % N_Z)
            right = (my_x, my_y, (my_z + 1) % N_Z)

            comm_ref[0, :, :] = out_ref[...]

            barrier_sem = pltpu.get_barrier_semaphore()
            for nbr in (left, right):
                pl.semaphore_signal(
                    barrier_sem, inc=1,
                    device_id=nbr, device_id_type=pl.DeviceIdType.MESH,
                )
            pl.semaphore_wait(barrier_sem, 2)

            for h in range(N_Z - 1):
                rdma = pltpu.make_async_remote_copy(
                    src_ref=comm_ref.at[h],
                    dst_ref=comm_ref.at[h + 1],
                    send_sem=send_sems.at[h],
                    recv_sem=recv_sems.at[h],
                    device_id=right,
                    device_id_type=pl.DeviceIdType.MESH,
                )
                rdma.start()
                rdma.wait()
                out_ref[...] += comm_ref[h + 1]

    return pl.pallas_call(
        body,
        grid=(nsteps,),
        out_shape=jax.ShapeDtypeStruct((2, d), jnp.float32),
        in_specs=[
            pl.BlockSpec((BLK, d), lambda i: (i, 0)),
            pl.BlockSpec((BLK, d), lambda i: (i, 0)),
            pl.BlockSpec(memory_space=pltpu.ANY),
        ],
        out_specs=pl.BlockSpec((2, d), lambda i: (0, 0)),
        scratch_shapes=[
            pltpu.VMEM((N_Z, 2, d), jnp.float32),
            pltpu.SemaphoreType.DMA((N_Z - 1,)),
            pltpu.SemaphoreType.DMA((N_Z - 1,)),
        ],
        compiler_params=pltpu.CompilerParams(
            dimension_semantics=("arbitrary",),
            collective_id=0,
        ),
    )(x, dy, gamma)


# baseline (device time: 33210 ns/iter reference)
import jax
import jax.numpy as jnp
from jax import lax
from jax.experimental import pallas as pl
from jax.experimental.pallas import tpu as pltpu

N_Z = 4
BLK = 512


def kernel(x, dy, gamma):
    m, d = x.shape
    nsteps = m // BLK

    def body(x_ref, dy_ref, gamma_ref, out_ref, comm_ref, send_sems, recv_sems):
        i = pl.program_id(0)

        @pl.when(i == 0)
        def _():
            out_ref[...] = jnp.zeros_like(out_ref)

        xv = x_ref[...]
        mu = jnp.mean(xv, axis=1, keepdims=True)
        xc = xv - mu
        var = jnp.mean(xc * xc, axis=1, keepdims=True)
        xhat = xc * lax.rsqrt(var + 1e-5)
        dyv = dy_ref[...]
        out_ref[0, :] += jnp.sum(dyv * xhat, axis=0)
        out_ref[1, :] += jnp.sum(dyv, axis=0)

        @pl.when(i == nsteps - 1)
        def _():
            my_x = lax.axis_index("x")
            my_y = lax.axis_index("y")
            my_z = lax.axis_index("z")
            left = (my_x, my_y, (my_z - 1) % N_Z)
            right = (my_x, my_y, (my_z + 1) % N_Z)

            comm_ref[0, :, :] = out_ref[...]

            barrier_sem = pltpu.get_barrier_semaphore()
            for nbr in (left, right):
                pl.semaphore_signal(
                    barrier_sem, inc=1,
                    device_id=nbr, device_id_type=pl.DeviceIdType.MESH,
                )
            pl.semaphore_wait(barrier_sem, 2)

            for h in range(N_Z - 1):
                rdma = pltpu.make_async_remote_copy(
                    src_ref=comm_ref.at[h],
                    dst_ref=comm_ref.at[h + 1],
                    send_sem=send_sems.at[h],
                    recv_sem=recv_sems.at[h],
                    device_id=right,
                    device_id_type=pl.DeviceIdType.MESH,
                )
                rdma.start()
                rdma.wait()
                out_ref[...] += comm_ref[h + 1]

    return pl.pallas_call(
        body,
        grid=(nsteps,),
        out_shape=jax.ShapeDtypeStruct((2, d), jnp.float32),
        in_specs=[
            pl.BlockSpec((BLK, d), lambda i: (i, 0)),
            pl.BlockSpec((BLK, d), lambda i: (i, 0)),
            pl.BlockSpec(memory_space=pl.ANY),
        ],
        out_specs=pl.BlockSpec((2, d), lambda i: (0, 0)),
        scratch_shapes=[
            pltpu.VMEM((N_Z, 2, d), jnp.float32),
            pltpu.SemaphoreType.DMA((N_Z - 1,)),
            pltpu.SemaphoreType.DMA((N_Z - 1,)),
        ],
        compiler_params=pltpu.CompilerParams(
            dimension_semantics=("arbitrary",),
            collective_id=0,
        ),
    )(x, dy, gamma)


# device time: 30886 ns/iter; 1.0752x vs baseline; 1.0752x over previous
import jax
import jax.numpy as jnp
from jax import lax
from jax.experimental import pallas as pl
from jax.experimental.pallas import tpu as pltpu

N_Z = 4
BLK = 512


def kernel(x, dy, gamma):
    m, d = x.shape
    nsteps = m // BLK

    def body(x_ref, dy_ref, gamma_ref, out_ref, comm_ref, send_sems, recv_sems):
        i = pl.program_id(0)
        my_x = lax.axis_index("x")
        my_y = lax.axis_index("y")
        my_z = lax.axis_index("z")
        partners = [(my_x, my_y, my_z ^ 1), (my_x, my_y, my_z ^ 2)]

        @pl.when(i == 0)
        def _():
            out_ref[...] = jnp.zeros_like(out_ref)
            barrier_sem = pltpu.get_barrier_semaphore()
            for nbr in partners:
                pl.semaphore_signal(
                    barrier_sem, inc=1,
                    device_id=nbr, device_id_type=pl.DeviceIdType.MESH,
                )
            pl.semaphore_wait(barrier_sem, 2)

        xv = x_ref[...]
        mu = jnp.mean(xv, axis=1, keepdims=True)
        xc = xv - mu
        var = jnp.mean(xc * xc, axis=1, keepdims=True)
        xhat = xc * lax.rsqrt(var + 1e-5)
        dyv = dy_ref[...]
        out_ref[0, :] += jnp.sum(dyv * xhat, axis=0)
        out_ref[1, :] += jnp.sum(dyv, axis=0)

        @pl.when(i == nsteps - 1)
        def _():
            for r in range(2):
                rdma = pltpu.make_async_remote_copy(
                    src_ref=out_ref,
                    dst_ref=comm_ref.at[r],
                    send_sem=send_sems.at[r],
                    recv_sem=recv_sems.at[r],
                    device_id=partners[r],
                    device_id_type=pl.DeviceIdType.MESH,
                )
                rdma.start()
                rdma.wait()
                out_ref[...] += comm_ref[r]

    return pl.pallas_call(
        body,
        grid=(nsteps,),
        out_shape=jax.ShapeDtypeStruct((2, d), jnp.float32),
        in_specs=[
            pl.BlockSpec((BLK, d), lambda i: (i, 0)),
            pl.BlockSpec((BLK, d), lambda i: (i, 0)),
            pl.BlockSpec(memory_space=pl.ANY),
        ],
        out_specs=pl.BlockSpec((2, d), lambda i: (0, 0)),
        scratch_shapes=[
            pltpu.VMEM((2, 2, d), jnp.float32),
            pltpu.SemaphoreType.DMA((2,)),
            pltpu.SemaphoreType.DMA((2,)),
        ],
        compiler_params=pltpu.CompilerParams(
            dimension_semantics=("arbitrary",),
            collective_id=0,
        ),
    )(x, dy, gamma)


# device time: 30469 ns/iter; 1.0900x vs baseline; 1.0137x over previous
import jax
import jax.numpy as jnp
from jax import lax
from jax.experimental import pallas as pl
from jax.experimental.pallas import tpu as pltpu

N_Z = 4
BLK = 512


def kernel(x, dy, gamma):
    m, d = x.shape
    nsteps = m // BLK

    def body(x_ref, dy_ref, gamma_ref, out_ref, comm_ref, send_sems, recv_sems):
        i = pl.program_id(0)
        my_x = lax.axis_index("x")
        my_y = lax.axis_index("y")
        my_z = lax.axis_index("z")
        partners = [(my_x, my_y, my_z ^ 1), (my_x, my_y, my_z ^ 2)]

        @pl.when(i == 0)
        def _():
            out_ref[...] = jnp.zeros_like(out_ref)
            barrier_sem = pltpu.get_barrier_semaphore()
            for nbr in partners:
                pl.semaphore_signal(
                    barrier_sem, inc=1,
                    device_id=nbr, device_id_type=pl.DeviceIdType.MESH,
                )
            pl.semaphore_wait(barrier_sem, 2)

        xv = x_ref[...]
        dyv = dy_ref[...]
        inv_d = 1.0 / d
        mu = jnp.sum(xv, axis=1, keepdims=True) * inv_d
        var = jnp.sum(xv * xv, axis=1, keepdims=True) * inv_d - mu * mu
        rstd = lax.rsqrt(var + 1e-5)
        t = dyv * xv
        dn = (((0,), (0,)), ((), ()))
        q = lax.dot_general(rstd, t, dn,
                            preferred_element_type=jnp.float32)
        w2 = jnp.concatenate([mu * rstd, jnp.ones_like(rstd)], axis=1)
        p = lax.dot_general(w2, dyv, dn,
                            preferred_element_type=jnp.float32)
        out_ref[0, :] += q[0, :] - p[0, :]
        out_ref[1, :] += p[1, :]

        @pl.when(i == nsteps - 1)
        def _():
            for r in range(2):
                rdma = pltpu.make_async_remote_copy(
                    src_ref=out_ref,
                    dst_ref=comm_ref.at[r],
                    send_sem=send_sems.at[r],
                    recv_sem=recv_sems.at[r],
                    device_id=partners[r],
                    device_id_type=pl.DeviceIdType.MESH,
                )
                rdma.start()
                rdma.wait()
                out_ref[...] += comm_ref[r]

    return pl.pallas_call(
        body,
        grid=(nsteps,),
        out_shape=jax.ShapeDtypeStruct((2, d), jnp.float32),
        in_specs=[
            pl.BlockSpec((BLK, d), lambda i: (i, 0)),
            pl.BlockSpec((BLK, d), lambda i: (i, 0)),
            pl.BlockSpec(memory_space=pl.ANY),
        ],
        out_specs=pl.BlockSpec((2, d), lambda i: (0, 0)),
        scratch_shapes=[
            pltpu.VMEM((2, 2, d), jnp.float32),
            pltpu.SemaphoreType.DMA((2,)),
            pltpu.SemaphoreType.DMA((2,)),
        ],
        compiler_params=pltpu.CompilerParams(
            dimension_semantics=("arbitrary",),
            collective_id=0,
        ),
    )(x, dy, gamma)
